# Initial kernel scaffold; baseline (speedup 1.0000x reference)
import jax
import jax.numpy as jnp
from jax.experimental import pallas as pl


def kernel(x):
    raise NotImplementedError("write your pallas kernel here")


# speedup vs baseline: 65.8070x; 65.8070x over previous
import numpy as np
import jax
import jax.numpy as jnp
from jax.experimental import pallas as pl
from jax.experimental.pallas import tpu as pltpu

_P = 0.1
_S = 1.0 - 2.0 * _P
_tmp = np.tile(np.array([_P, _S, _P], dtype=np.float32), (3, 3, 1))
_tmp[2, 1] = np.array([0.0, _S / (_P + _S), _P / (_P + _S)], dtype=np.float32)
_LOGITS = np.where(_tmp > 0, np.log(np.maximum(_tmp, 1e-30)), -1e30).astype(np.float32)
_LN = _LOGITS[0, 0]
_LS = _LOGITS[2, 1]


def _chain_kernel(g_ref, d_ref, a_scr, c_scr):
    n1 = g_ref.shape[1]
    g0 = g_ref[0]
    g1 = g_ref[1]
    g2 = g_ref[2]
    v0 = g0 + _LN[0]
    v1 = g1 + _LN[1]
    v2 = g2 + _LN[2]
    a = jnp.where(v2 > jnp.maximum(v0, v1), 2, jnp.where(v1 > v0, 1, 0))
    w1 = g1 + _LS[1]
    w2 = g2 + _LS[2]
    c = jnp.where(w2 > w1, 2, 1)
    a_scr[...] = a.astype(jnp.int32)
    c_scr[...] = c.astype(jnp.int32)

    bsz = d_ref.shape[1]
    d_ref[0:1, :] = jnp.ones((1, bsz), jnp.int32)
    ones = d_ref[0:1, :]

    def body(t, carry):
        p2, p1 = carry
        a_row = a_scr[pl.ds(t, 1), :]
        c_row = c_scr[pl.ds(t, 1), :]
        samp = jnp.where((p1 == 2) & (p2 == 1), c_row, a_row)
        d_ref[pl.ds(t + 1, 1), :] = samp
        return (p1, samp)

    jax.lax.fori_loop(0, n1, body, (ones, ones))


def _select_kernel(d_ref, x_ref, o_ref):
    d = d_ref[0]
    n = o_ref.shape[2]
    x0 = x_ref[0, :, pl.ds(0, n)]
    x1 = x_ref[0, :, pl.ds(1, n)]
    x2 = x_ref[0, :, pl.ds(2, n)]
    o_ref[0] = jnp.where(d == 0, x0, jnp.where(d == 1, x1, x2))


def kernel(x):
    B, I, T = x.shape
    n_win = T - 2
    n1 = n_win - 1

    keys = jax.random.split(jax.random.key(42), n1)
    g = jax.vmap(lambda k: jax.random.gumbel(k, (B, 3), jnp.float32))(keys)
    gp = g.transpose(2, 0, 1)

    d = pl.pallas_call(
        _chain_kernel,
        out_shape=jax.ShapeDtypeStruct((n_win, B), jnp.int32),
        scratch_shapes=[
            pltpu.VMEM((n1, B), jnp.int32),
            pltpu.VMEM((n1, B), jnp.int32),
        ],
    )(gp)

    d3 = d.T.reshape(B, 1, n_win)

    out = pl.pallas_call(
        _select_kernel,
        grid=(B,),
        in_specs=[
            pl.BlockSpec((1, 1, n_win), lambda b: (b, 0, 0)),
            pl.BlockSpec((1, I, T), lambda b: (b, 0, 0)),
        ],
        out_specs=pl.BlockSpec((1, I, n_win), lambda b: (b, 0, 0)),
        out_shape=jax.ShapeDtypeStruct((B, I, n_win), x.dtype),
    )(d3, x)
    return out
